# Initial kernel scaffold; baseline (speedup 1.0000x reference)
import jax
import jax.numpy as jnp
from jax.experimental import pallas as pl


def kernel(features, W_conv, b_conv, W_loc, b_loc, W_score, b_score, img_size, scale):
    raise NotImplementedError("write your pallas kernel here")


# speedup vs baseline: 15.2089x; 15.2089x over previous
import functools

import jax
import jax.numpy as jnp
import numpy as np
from jax import lax
from jax.experimental import pallas as pl
from jax.experimental.pallas import tpu as pltpu

_SCALES = [8, 16, 32]
_RATIOS = [0.5, 1.0, 2.0]
_N_ANCHOR = 9
_FH = 50
_FW = 50
_FEAT_STRIDE = 16
_NMS_THRESH = 0.7
_N_PRE = 6000
_N_POST = 300
_MIN_SIZE = 16

_NPAD = 6144
_NBLK = 47
_BLK = 128
_CHUNK = 2048


def _anchors_np():
    base = float(_FEAT_STRIDE)
    ab = []
    for r in _RATIOS:
        for s in _SCALES:
            h = base * s * np.sqrt(r)
            w = base * s * np.sqrt(1.0 / r)
            ab.append([base / 2 - h / 2, base / 2 - w / 2,
                       base / 2 + h / 2, base / 2 + w / 2])
    ab = np.asarray(ab, np.float32)
    sy = np.arange(_FH, dtype=np.float32) * _FEAT_STRIDE
    sx = np.arange(_FW, dtype=np.float32) * _FEAT_STRIDE
    yy, xx = np.meshgrid(sy, sx, indexing='ij')
    shifts = np.stack([yy.ravel(), xx.ravel(), yy.ravel(), xx.ravel()], 1)
    return (shifts[:, None, :] + ab[None, :, :]).reshape(-1, 4)


def _nms_select_body(y1c, x1c, y2c, x2c, y1r, x1r, y2r, x2r, fin, o_ref, t_ref):
    gio = lax.broadcasted_iota(jnp.int32, (1, _NPAD), 1)
    y1rv = y1r[...]
    x1rv = x1r[...]
    y2rv = y2r[...]
    x2rv = x2r[...]
    ar_row = (y2rv - y1rv) * (x2rv - x1rv)

    sup = jnp.zeros((1, _NPAD), jnp.float32)
    kept_blocks = []
    lio = lax.broadcasted_iota(jnp.int32, (1, _BLK), 1)

    for b in range(_NBLK):
        r0 = b * _BLK
        y1b = y1c[r0:r0 + _BLK, :]
        x1b = x1c[r0:r0 + _BLK, :]
        y2b = y2c[r0:r0 + _BLK, :]
        x2b = x2c[r0:r0 + _BLK, :]
        ab_col = (y2b - y1b) * (x2b - x1b)
        chunks = []
        for c0 in range(0, _NPAD, _CHUNK):
            c1 = c0 + _CHUNK
            iy1 = jnp.maximum(y1b, y1rv[:, c0:c1])
            ix1 = jnp.maximum(x1b, x1rv[:, c0:c1])
            iy2 = jnp.minimum(y2b, y2rv[:, c0:c1])
            ix2 = jnp.minimum(x2b, x2rv[:, c0:c1])
            inter = jnp.clip(iy2 - iy1, 0.0) * jnp.clip(ix2 - ix1, 0.0)
            iou = inter / (ab_col + ar_row[:, c0:c1] - inter + 1e-9)
            chunks.append((iou > _NMS_THRESH).astype(jnp.float32))
        conf = jnp.concatenate(chunks, axis=1)
        t_ref[...] = conf[:, r0:r0 + _BLK]

        act0 = 1.0 - sup[:, r0:r0 + _BLK]

        def body(t, act):
            trow = t_ref[pl.ds(t, 1), :]
            kt = jnp.sum(act * (lio == t).astype(jnp.float32))
            gt = (lio > t).astype(jnp.float32)
            return act * (1.0 - kt * trow * gt)

        actf = lax.fori_loop(0, _BLK, body, act0)
        kept_blocks.append(actf)
        k8 = jnp.broadcast_to(actf, (8, _BLK))
        m8 = lax.dot_general(k8, conf, (((1,), (0,)), ((), ())),
                             preferred_element_type=jnp.float32)
        hit = ((m8[0:1, :] > 0.0) & (gio >= r0 + _BLK)).astype(jnp.float32)
        sup = jnp.maximum(sup, hit)

    kept_blocks.append(jnp.zeros((1, _NPAD - _NBLK * _BLK), jnp.float32))
    keep = jnp.concatenate(kept_blocks, axis=1)
    valid = (gio < _N_PRE).astype(jnp.float32)
    finv = fin[...]
    grp_a = keep * finv * valid
    grp_b = valid * (1.0 - keep * finv)
    ut = (lax.broadcasted_iota(jnp.int32, (_BLK, _BLK), 0)
          <= lax.broadcasted_iota(jnp.int32, (_BLK, _BLK), 1)).astype(jnp.float32)

    def excl_prefix(v):
        parts = []
        off = jnp.zeros((1, 1), jnp.float32)
        for c in range(_NPAD // _BLK):
            vc = v[:, c * _BLK:(c + 1) * _BLK]
            p8 = lax.dot_general(jnp.broadcast_to(vc, (8, _BLK)), ut,
                                 (((1,), (0,)), ((), ())),
                                 preferred_element_type=jnp.float32)
            pc = p8[0:1, :]
            parts.append(pc - vc + off)
            off = off + pc[:, _BLK - 1:_BLK]
        return parts, off

    ra, tot_a = excl_prefix(grp_a)
    rb, _ = excl_prefix(grp_b)
    big = jnp.float32(1.0e9)
    rank = [jnp.where(grp_a[:, c * _BLK:(c + 1) * _BLK] > 0.0, ra[c],
                      jnp.where(grp_b[:, c * _BLK:(c + 1) * _BLK] > 0.0,
                                rb[c] + tot_a, big))
            for c in range(_NPAD // _BLK)]

    rcio = lax.broadcasted_iota(jnp.int32, (1, _BLK), 1).astype(jnp.float32)
    out_chunks = []
    for rc in range(3):
        acc = jnp.zeros((1, _BLK), jnp.float32)
        target0 = jnp.float32(rc * _BLK)
        for c in range(_NPAD // _BLK):
            rct = jnp.transpose(rank[c], (1, 0))
            oh = (rct == (rcio + target0)).astype(jnp.float32)
            l8 = jnp.broadcast_to(lio.astype(jnp.float32), (8, _BLK))
            ones8 = jnp.ones((8, _BLK), jnp.float32)
            dl = lax.dot_general(l8, oh, (((1,), (0,)), ((), ())),
                                 preferred_element_type=jnp.float32)[0:1, :]
            dc = lax.dot_general(ones8, oh, (((1,), (0,)), ((), ())),
                                 preferred_element_type=jnp.float32)[0:1, :]
            acc = acc + dl + jnp.float32(c * _BLK) * dc
        out_chunks.append(jnp.broadcast_to(acc, (8, _BLK)))
    o_ref[...] = jnp.concatenate(out_chunks, axis=1)


@functools.partial(jax.jit, static_argnames=())
def _nms_select(b6, fin_f):
    cols = [jnp.pad(b6[:, i], (0, _NPAD - _N_PRE)).reshape(_NPAD, 1)
            for i in range(4)]
    rows = [jnp.pad(b6[:, i], (0, _NPAD - _N_PRE)).reshape(1, _NPAD)
            for i in range(4)]
    fin = jnp.pad(fin_f, (0, _NPAD - _N_PRE)).reshape(1, _NPAD)
    out = pl.pallas_call(
        _nms_select_body,
        out_shape=jax.ShapeDtypeStruct((8, 384), jnp.float32),
        scratch_shapes=[pltpu.VMEM((_BLK, _BLK), jnp.float32)],
    )(*cols, *rows, fin)
    return out[0, :_N_POST].astype(jnp.int32)


def kernel(features, W_conv, b_conv, W_loc, b_loc, W_score, b_score,
           img_size, scale):
    def conv(x, w, b):
        y = lax.conv_general_dilated(x, w, (1, 1), 'SAME',
                                     dimension_numbers=('NCHW', 'OIHW', 'NCHW'))
        return y + b.reshape(1, -1, 1, 1)

    h = jax.nn.relu(conv(features, W_conv, b_conv))
    locmap = conv(h, W_loc, b_loc)
    smap = conv(h, W_score, b_score)
    n, _, fh, fw = locmap.shape
    loc = jnp.transpose(locmap, (0, 2, 3, 1)).reshape(n, -1, 4)[0]
    score = jnp.transpose(smap, (0, 2, 3, 1))
    sm = jax.nn.softmax(score.reshape(n, fh, fw, _N_ANCHOR, 2), axis=4)
    fg = sm[..., 1].reshape(n, -1)[0]

    anchor = jnp.asarray(_anchors_np())
    ah = anchor[:, 2] - anchor[:, 0]
    aw = anchor[:, 3] - anchor[:, 1]
    acy = anchor[:, 0] + 0.5 * ah
    acx = anchor[:, 1] + 0.5 * aw
    ncy = loc[:, 0] * ah + acy
    ncx = loc[:, 1] * aw + acx
    nh = jnp.exp(loc[:, 2]) * ah
    nw = jnp.exp(loc[:, 3]) * aw
    roi = jnp.stack([ncy - 0.5 * nh, ncx - 0.5 * nw,
                     ncy + 0.5 * nh, ncx + 0.5 * nw], axis=1)
    roi = jnp.clip(roi, 0.0, jnp.asarray(img_size).astype(jnp.float32))

    hs = roi[:, 2] - roi[:, 0]
    ws = roi[:, 3] - roi[:, 1]
    ms = _MIN_SIZE * jnp.asarray(scale).astype(jnp.float32)
    score2 = jnp.where((hs >= ms) & (ws >= ms), fg, -jnp.inf)

    top_s, top_i = lax.top_k(score2, _N_PRE)
    b6 = roi[top_i]
    fin_f = (top_s > -jnp.inf).astype(jnp.float32)

    idx2 = _nms_select(b6, fin_f)
    return b6[idx2]


# speedup vs baseline: 56.7668x; 3.7325x over previous
import functools

import jax
import jax.numpy as jnp
import numpy as np
from jax import lax
from jax.experimental import pallas as pl

_SCALES = [8, 16, 32]
_RATIOS = [0.5, 1.0, 2.0]
_N_ANCHOR = 9
_FH = 50
_FW = 50
_FEAT_STRIDE = 16
_NMS_THRESH = 0.7
_N_PRE = 6000
_N_POST = 300
_MIN_SIZE = 16

_NPAD = 6144
_NBLK = 47
_BLK = 128
_CHUNK = 1024


def _anchors_np():
    base = float(_FEAT_STRIDE)
    ab = []
    for r in _RATIOS:
        for s in _SCALES:
            h = base * s * np.sqrt(r)
            w = base * s * np.sqrt(1.0 / r)
            ab.append([base / 2 - h / 2, base / 2 - w / 2,
                       base / 2 + h / 2, base / 2 + w / 2])
    ab = np.asarray(ab, np.float32)
    sy = np.arange(_FH, dtype=np.float32) * _FEAT_STRIDE
    sx = np.arange(_FW, dtype=np.float32) * _FEAT_STRIDE
    yy, xx = np.meshgrid(sy, sx, indexing='ij')
    shifts = np.stack([yy.ravel(), xx.ravel(), yy.ravel(), xx.ravel()], 1)
    return (shifts[:, None, :] + ab[None, :, :]).reshape(-1, 4)


def _nms_select_body(y1c, x1c, y2c, x2c, y1r, x1r, y2r, x2r, fin, o_ref):
    gio = lax.broadcasted_iota(jnp.int32, (1, _NPAD), 1)
    y1rv = y1r[...]
    x1rv = x1r[...]
    y2rv = y2r[...]
    x2rv = x2r[...]
    ar_row = (y2rv - y1rv) * (x2rv - x1rv)

    sup = jnp.zeros((1, _NPAD), jnp.float32)
    kept_blocks = []
    lio = lax.broadcasted_iota(jnp.int32, (1, _BLK), 1)
    upper_strict = (lax.broadcasted_iota(jnp.int32, (_BLK, _BLK), 0)
                    < lax.broadcasted_iota(jnp.int32, (_BLK, _BLK), 1)
                    ).astype(jnp.float32)

    for b in range(_NBLK):
        r0 = b * _BLK
        cbase = (r0 // _CHUNK) * _CHUNK
        y1b = y1c[r0:r0 + _BLK, :]
        x1b = x1c[r0:r0 + _BLK, :]
        y2b = y2c[r0:r0 + _BLK, :]
        x2b = x2c[r0:r0 + _BLK, :]
        ab_col = (y2b - y1b) * (x2b - x1b)
        chunks = []
        for c0 in range(cbase, _NPAD, _CHUNK):
            c1 = c0 + _CHUNK
            iy1 = jnp.maximum(y1b, y1rv[:, c0:c1])
            ix1 = jnp.maximum(x1b, x1rv[:, c0:c1])
            iy2 = jnp.minimum(y2b, y2rv[:, c0:c1])
            ix2 = jnp.minimum(x2b, x2rv[:, c0:c1])
            inter = jnp.clip(iy2 - iy1, 0.0) * jnp.clip(ix2 - ix1, 0.0)
            iou = inter / (ab_col + ar_row[:, c0:c1] - inter + 1e-9)
            chunks.append((iou > _NMS_THRESH).astype(jnp.float32))
        conf = jnp.concatenate(chunks, axis=1)
        tup = conf[:, r0 - cbase:r0 - cbase + _BLK] * upper_strict

        act0 = 1.0 - sup[:, r0:r0 + _BLK]

        def cond(c):
            return c[1] > 0.0

        def sweep(c):
            k, _ = c
            mv = lax.dot_general(jnp.broadcast_to(k, (8, _BLK)), tup,
                                 (((1,), (0,)), ((), ())),
                                 preferred_element_type=jnp.float32)[0:1, :]
            k_new = act0 * (1.0 - (mv > 0.0).astype(jnp.float32))
            return (k_new, jnp.sum(jnp.abs(k_new - k)))

        actf, _ = lax.while_loop(cond, sweep, (act0, jnp.float32(1.0)))
        kept_blocks.append(actf)
        k8 = jnp.broadcast_to(actf, (8, _BLK))
        m8 = lax.dot_general(k8, conf, (((1,), (0,)), ((), ())),
                             preferred_element_type=jnp.float32)
        hit = ((m8[0:1, :] > 0.0)
               & (gio[:, cbase:] >= r0 + _BLK)).astype(jnp.float32)
        if cbase:
            hit = jnp.concatenate(
                [jnp.zeros((1, cbase), jnp.float32), hit], axis=1)
        sup = jnp.maximum(sup, hit)

    kept_blocks.append(jnp.zeros((1, _NPAD - _NBLK * _BLK), jnp.float32))
    keep = jnp.concatenate(kept_blocks, axis=1)
    valid = (gio < _N_PRE).astype(jnp.float32)
    finv = fin[...]
    grp_a = keep * finv * valid
    grp_b = valid * (1.0 - keep * finv)
    ut = (lax.broadcasted_iota(jnp.int32, (_BLK, _BLK), 0)
          <= lax.broadcasted_iota(jnp.int32, (_BLK, _BLK), 1)).astype(jnp.float32)

    def excl_prefix(v):
        parts = []
        off = jnp.zeros((1, 1), jnp.float32)
        for c in range(_NPAD // _BLK):
            vc = v[:, c * _BLK:(c + 1) * _BLK]
            p8 = lax.dot_general(jnp.broadcast_to(vc, (8, _BLK)), ut,
                                 (((1,), (0,)), ((), ())),
                                 preferred_element_type=jnp.float32)
            pc = p8[0:1, :]
            parts.append(pc - vc + off)
            off = off + pc[:, _BLK - 1:_BLK]
        return parts, off

    ra, tot_a = excl_prefix(grp_a)
    rb, _ = excl_prefix(grp_b)
    big = jnp.float32(1.0e9)
    rank = [jnp.where(grp_a[:, c * _BLK:(c + 1) * _BLK] > 0.0, ra[c],
                      jnp.where(grp_b[:, c * _BLK:(c + 1) * _BLK] > 0.0,
                                rb[c] + tot_a, big))
            for c in range(_NPAD // _BLK)]

    rcio = lax.broadcasted_iota(jnp.int32, (1, _BLK), 1).astype(jnp.float32)
    out_chunks = []
    for rc in range(3):
        acc = jnp.zeros((1, _BLK), jnp.float32)
        target0 = jnp.float32(rc * _BLK)
        for c in range(_NPAD // _BLK):
            rct = jnp.transpose(rank[c], (1, 0))
            oh = (rct == (rcio + target0)).astype(jnp.float32)
            l8 = jnp.broadcast_to(lio.astype(jnp.float32), (8, _BLK))
            ones8 = jnp.ones((8, _BLK), jnp.float32)
            dl = lax.dot_general(l8, oh, (((1,), (0,)), ((), ())),
                                 preferred_element_type=jnp.float32)[0:1, :]
            dc = lax.dot_general(ones8, oh, (((1,), (0,)), ((), ())),
                                 preferred_element_type=jnp.float32)[0:1, :]
            acc = acc + dl + jnp.float32(c * _BLK) * dc
        out_chunks.append(jnp.broadcast_to(acc, (8, _BLK)))
    o_ref[...] = jnp.concatenate(out_chunks, axis=1)


@functools.partial(jax.jit, static_argnames=())
def _nms_select(b6, fin_f):
    cols = [jnp.pad(b6[:, i], (0, _NPAD - _N_PRE)).reshape(_NPAD, 1)
            for i in range(4)]
    rows = [jnp.pad(b6[:, i], (0, _NPAD - _N_PRE)).reshape(1, _NPAD)
            for i in range(4)]
    fin = jnp.pad(fin_f, (0, _NPAD - _N_PRE)).reshape(1, _NPAD)
    out = pl.pallas_call(
        _nms_select_body,
        out_shape=jax.ShapeDtypeStruct((8, 384), jnp.float32),
    )(*cols, *rows, fin)
    return out[0, :_N_POST].astype(jnp.int32)


def kernel(features, W_conv, b_conv, W_loc, b_loc, W_score, b_score,
           img_size, scale):
    def conv(x, w, b):
        y = lax.conv_general_dilated(x, w, (1, 1), 'SAME',
                                     dimension_numbers=('NCHW', 'OIHW', 'NCHW'))
        return y + b.reshape(1, -1, 1, 1)

    h = jax.nn.relu(conv(features, W_conv, b_conv))
    locmap = conv(h, W_loc, b_loc)
    smap = conv(h, W_score, b_score)
    n, _, fh, fw = locmap.shape
    loc = jnp.transpose(locmap, (0, 2, 3, 1)).reshape(n, -1, 4)[0]
    score = jnp.transpose(smap, (0, 2, 3, 1))
    sm = jax.nn.softmax(score.reshape(n, fh, fw, _N_ANCHOR, 2), axis=4)
    fg = sm[..., 1].reshape(n, -1)[0]

    anchor = jnp.asarray(_anchors_np())
    ah = anchor[:, 2] - anchor[:, 0]
    aw = anchor[:, 3] - anchor[:, 1]
    acy = anchor[:, 0] + 0.5 * ah
    acx = anchor[:, 1] + 0.5 * aw
    ncy = loc[:, 0] * ah + acy
    ncx = loc[:, 1] * aw + acx
    nh = jnp.exp(loc[:, 2]) * ah
    nw = jnp.exp(loc[:, 3]) * aw
    roi = jnp.stack([ncy - 0.5 * nh, ncx - 0.5 * nw,
                     ncy + 0.5 * nh, ncx + 0.5 * nw], axis=1)
    roi = jnp.clip(roi, 0.0, jnp.asarray(img_size).astype(jnp.float32))

    hs = roi[:, 2] - roi[:, 0]
    ws = roi[:, 3] - roi[:, 1]
    ms = _MIN_SIZE * jnp.asarray(scale).astype(jnp.float32)
    score2 = jnp.where((hs >= ms) & (ws >= ms), fg, -jnp.inf)

    top_s, top_i = lax.top_k(score2, _N_PRE)
    b6 = roi[top_i]
    fin_f = (top_s > -jnp.inf).astype(jnp.float32)

    idx2 = _nms_select(b6, fin_f)
    return b6[idx2]
